# Initial kernel scaffold; baseline (speedup 1.0000x reference)
import jax
import jax.numpy as jnp
from jax.experimental import pallas as pl


def kernel(x, edge_index, W0, al0, ar0, b0, W1, al1, ar1, b1, W2, al2, ar2, b2):
    raise NotImplementedError("write your pallas kernel here")


# speedup vs baseline: 20.2211x; 20.2211x over previous
import functools

import jax
import jax.numpy as jnp
from jax import lax
from jax.experimental import pallas as pl
from jax.experimental.pallas import tpu as pltpu
from jax.experimental.pallas import tpu_sc as plsc

N = 10000
E = 320000
F = 128
H = 4
DH = 32
C = 40

NC = 2
NS = 16
NW = NC * NS
EPW = E // NW
CH = 80
NCH = EPW // CH
RPT = N // NS

TW01 = 144
TW2 = 48

BN = 1000


def _iota16():
    return lax.broadcasted_iota(jnp.int32, (16,), 0)


def _c16(v):
    return jnp.full((16,), v, jnp.int32)


def _make_edge_kernel(tbl_w, num_heads, dh):
    fd = num_heads * dh
    mesh = plsc.VectorSubcoreMesh(core_axis_name="c", subcore_axis_name="s",
                                  num_cores=NC, num_subcores=NS)

    @functools.partial(
        pl.kernel,
        out_type=jax.ShapeDtypeStruct((NC, N, tbl_w), jnp.float32),
        mesh=mesh,
        scratch_types=[
            pltpu.VMEM((CH,), jnp.int32),
            pltpu.VMEM((CH,), jnp.int32),
            pltpu.VMEM((CH, 16), jnp.float32),
            pltpu.VMEM((CH, tbl_w), jnp.float32),
            pltpu.VMEM_SHARED((N, tbl_w), jnp.float32),
            pltpu.SemaphoreType.DMA,
            pltpu.SemaphoreType.DMA,
            pltpu.SemaphoreType.DMA,
        ],
        compiler_params=pltpu.CompilerParams(use_tc_tiling_on_sc=False,
                                             needs_layout_passes=False),
    )
    def edge_k(tbl_hbm, er_hbm, src_hbm, dst_hbm, out_hbm,
               src_c, dst_c, er_v, rows_v, acc_s, gsem, esem, ssem):
        cid = lax.axis_index("c")
        sid = lax.axis_index("s")
        wid = cid * NS + sid

        zero16 = jnp.zeros((16,), jnp.float32)

        def zrow(i, carry):
            for cc in range(tbl_w // 16):
                rows_v[i, pl.ds(cc * 16, 16)] = zero16
            return carry

        lax.fori_loop(0, CH, zrow, 0)

        base = sid * RPT
        nfull = RPT // CH
        rem = RPT - nfull * CH
        for k in range(nfull):
            pltpu.sync_copy(rows_v, acc_s.at[pl.ds(base + k * CH, CH)])
        pltpu.sync_copy(rows_v.at[pl.ds(0, rem)],
                        acc_s.at[pl.ds(base + nfull * CH, rem)])

        plsc.subcore_barrier()

        def chunk_body(j, carry):
            pltpu.sync_copy(src_hbm.at[wid, j], src_c)
            pltpu.sync_copy(dst_hbm.at[wid, j], dst_c)
            pltpu.async_copy(tbl_hbm.at[src_c], rows_v, gsem).wait()
            pltpu.async_copy(er_hbm.at[dst_c], er_v, esem).wait()
            for g in range(CH // 16):
                eids = _iota16() + g * 16
                for h in range(num_heads):
                    ecol = _c16(fd + h)
                    el = plsc.load_gather(rows_v, [eids, ecol])
                    erv = plsc.load_gather(er_v, [eids, _c16(h)])
                    e = el + erv
                    e = jnp.where(e > 0, e, 0.2 * e)
                    ex = jnp.exp(e)
                    for cc in range(h * dh, (h + 1) * dh):
                        cv = _c16(cc)
                        v = plsc.load_gather(rows_v, [eids, cv])
                        plsc.store_scatter(rows_v, [eids, cv], v * ex)
                    plsc.store_scatter(rows_v, [eids, ecol], ex)
            pltpu.async_copy(rows_v, acc_s.at[dst_c], ssem, add=True).wait()
            return carry

        lax.fori_loop(0, NCH, chunk_body, 0)

        plsc.subcore_barrier()
        pltpu.sync_copy(acc_s.at[pl.ds(base, RPT)],
                        out_hbm.at[cid, pl.ds(base, RPT)])

    return edge_k


_edge_cache = {}


def _edge_kernel(tbl_w, num_heads, dh):
    key = (tbl_w, num_heads, dh)
    if key not in _edge_cache:
        _edge_cache[key] = _make_edge_kernel(tbl_w, num_heads, dh)
    return _edge_cache[key]


def _feed0_body(x_ref, wa_ref, ar_ref, tbl_ref, er_ref):
    xb = x_ref[...]
    tbl_ref[...] = jnp.dot(xb, wa_ref[...], preferred_element_type=jnp.float32)
    er_ref[...] = jnp.dot(xb, ar_ref[...], preferred_element_type=jnp.float32)


def _combine_feed_body(a_ref, b_ref, r_ref, bias_ref, wa_ref, ar_ref,
                       tbl_ref, er_ref):
    s = a_ref[...] + b_ref[...]
    den = jnp.dot(s[:, F:F + H], r_ref[...], preferred_element_type=jnp.float32)
    hb = s[:, :F] / jnp.maximum(den, 1e-9) + bias_ref[...]
    tbl_ref[...] = jnp.dot(hb, wa_ref[...], preferred_element_type=jnp.float32)
    er_ref[...] = jnp.dot(hb, ar_ref[...], preferred_element_type=jnp.float32)


def _final_body(a_ref, b_ref, bias_ref, o_ref):
    s = a_ref[...] + b_ref[...]
    den = jnp.maximum(s[:, C:C + 1], 1e-9)
    o_ref[...] = s[:, :C] / den + bias_ref[...]


def _row_spec(w):
    return pl.BlockSpec((BN, w), lambda i: (i, 0))


def _rep_spec(shape):
    return pl.BlockSpec(shape, lambda i: (0, 0))


def _feed0(x, wa, ar):
    return pl.pallas_call(
        _feed0_body,
        grid=(N // BN,),
        in_specs=[_row_spec(F), _rep_spec((F, TW01)), _rep_spec((F, 16))],
        out_specs=[_row_spec(TW01), _row_spec(16)],
        out_shape=[jax.ShapeDtypeStruct((N, TW01), jnp.float32),
                   jax.ShapeDtypeStruct((N, 16), jnp.float32)],
    )(x, wa, ar)


def _combine_feed(acc, r4, bias, wa, ar, tw):
    return pl.pallas_call(
        _combine_feed_body,
        grid=(N // BN,),
        in_specs=[_row_spec(TW01), _row_spec(TW01), _rep_spec((H, F)),
                  _rep_spec((1, F)), _rep_spec((F, tw)), _rep_spec((F, 16))],
        out_specs=[_row_spec(tw), _row_spec(16)],
        out_shape=[jax.ShapeDtypeStruct((N, tw), jnp.float32),
                   jax.ShapeDtypeStruct((N, 16), jnp.float32)],
    )(acc[0], acc[1], r4, bias, wa, ar)


def _final(acc, bias):
    return pl.pallas_call(
        _final_body,
        grid=(N // BN,),
        in_specs=[_row_spec(TW2), _row_spec(TW2), _rep_spec((1, C))],
        out_specs=_row_spec(C),
        out_shape=jax.ShapeDtypeStruct((N, C), jnp.float32),
    )(acc[0], acc[1], bias)


def _attn_mat(al, num_heads, dh):
    eye = jnp.eye(num_heads, dtype=al.dtype)
    return (al[:, :, None] * eye[:, None, :]).reshape(num_heads * dh, num_heads)


def kernel(x, edge_index, W0, al0, ar0, b0, W1, al1, ar1, b1, W2, al2, ar2, b2):
    src3 = edge_index[0].reshape(NW, NCH, CH)
    dst3 = edge_index[1].reshape(NW, NCH, CH)

    zpad01 = jnp.zeros((F, TW01 - F - H), jnp.float32)
    wa0 = jnp.concatenate([W0, W0 @ _attn_mat(al0, H, DH), zpad01], axis=1)
    wr0 = jnp.concatenate([W0 @ _attn_mat(ar0, H, DH), jnp.zeros((F, 12), jnp.float32)], axis=1)
    wa1 = jnp.concatenate([W1, W1 @ _attn_mat(al1, H, DH), zpad01], axis=1)
    wr1 = jnp.concatenate([W1 @ _attn_mat(ar1, H, DH), jnp.zeros((F, 12), jnp.float32)], axis=1)
    wa2 = jnp.concatenate([W2, W2 @ al2.T, jnp.zeros((F, TW2 - C - 1), jnp.float32)],
                          axis=1)
    wr2 = jnp.concatenate([W2 @ ar2.T, jnp.zeros((F, 15), jnp.float32)], axis=1)
    r4 = jnp.kron(jnp.eye(H, dtype=jnp.float32), jnp.ones((1, DH), jnp.float32))

    edge01 = _edge_kernel(TW01, H, DH)
    edge2 = _edge_kernel(TW2, 1, C)

    tbl0, er0 = _feed0(x, wa0, wr0)
    acc0 = edge01(tbl0, er0, src3, dst3)
    tbl1, er1 = _combine_feed(acc0, r4, b0.reshape(1, F), wa1, wr1, TW01)
    acc1 = edge01(tbl1, er1, src3, dst3)
    tbl2, er2 = _combine_feed(acc1, r4, b1.reshape(1, F), wa2, wr2, TW2)
    acc2 = edge2(tbl2, er2, src3, dst3)
    return _final(acc2, b2.reshape(1, C))


# speedup vs baseline: 47.6868x; 2.3583x over previous
import functools

import jax
import jax.numpy as jnp
from jax import lax
from jax.experimental import pallas as pl
from jax.experimental.pallas import tpu as pltpu
from jax.experimental.pallas import tpu_sc as plsc

N = 10000
E = 320000
F = 128
H = 4
DH = 32
C = 40

NC = 2
NS = 16
NW = NC * NS
EPW = E // NW
CH = 80
NCH = EPW // CH
RPT = N // NS

TW01 = 144
TW2 = 48

BN = 1000


def _iota16():
    return lax.broadcasted_iota(jnp.int32, (16,), 0)


def _c16(v):
    return jnp.full((16,), v, jnp.int32)


def _make_edge_kernel(tbl_w, num_heads, dh):
    fd = num_heads * dh
    mesh = plsc.VectorSubcoreMesh(core_axis_name="c", subcore_axis_name="s",
                                  num_cores=NC, num_subcores=NS)

    @functools.partial(
        pl.kernel,
        out_type=jax.ShapeDtypeStruct((NC, N, tbl_w), jnp.float32),
        mesh=mesh,
        scratch_types=[
            pltpu.VMEM((CH,), jnp.int32),
            pltpu.VMEM((CH,), jnp.int32),
            pltpu.VMEM((CH, 16), jnp.float32),
            pltpu.VMEM((CH, tbl_w), jnp.float32),
            pltpu.VMEM((CH,), jnp.int32),
            pltpu.VMEM((CH,), jnp.int32),
            pltpu.VMEM((CH, 16), jnp.float32),
            pltpu.VMEM((CH, tbl_w), jnp.float32),
            pltpu.VMEM_SHARED((N, tbl_w), jnp.float32),
            pltpu.SemaphoreType.DMA,
            pltpu.SemaphoreType.DMA,
            pltpu.SemaphoreType.DMA,
            pltpu.SemaphoreType.DMA,
            pltpu.SemaphoreType.DMA,
        ],
        compiler_params=pltpu.CompilerParams(use_tc_tiling_on_sc=False,
                                             needs_layout_passes=False),
    )
    def edge_k(tbl_hbm, er_hbm, src_hbm, dst_hbm, out_hbm,
               src_a, dst_a, er_a, rows_a, src_b, dst_b, er_b, rows_b,
               acc_s, gsa, esa, gsb, esb, ssem):
        cid = lax.axis_index("c")
        sid = lax.axis_index("s")
        wid = cid * NS + sid
        bufs = ((src_a, dst_a, er_a, rows_a, gsa, esa),
                (src_b, dst_b, er_b, rows_b, gsb, esb))

        zero16 = jnp.zeros((16,), jnp.float32)

        def zrow(i, carry):
            for cc in range(tbl_w // 16):
                rows_a[i, pl.ds(cc * 16, 16)] = zero16
            return carry

        lax.fori_loop(0, CH, zrow, 0)

        base = sid * RPT
        nfull = RPT // CH
        rem = RPT - nfull * CH
        for k in range(nfull):
            pltpu.sync_copy(rows_a, acc_s.at[pl.ds(base + k * CH, CH)])
        pltpu.sync_copy(rows_a.at[pl.ds(0, rem)],
                        acc_s.at[pl.ds(base + nfull * CH, rem)])

        plsc.subcore_barrier()

        if num_heads == 4:
            ecol0 = fd
            moff = 0
        else:
            ecol0 = fd - 8
            moff = 8
        lanes = _iota16()
        hmask = jnp.where((lanes >= moff) & (lanes < moff + num_heads),
                          1.0, 0.0).astype(jnp.float32)

        def issue(j, b):
            src_c, dst_c, er_v, rows_v, gsem, esem = bufs[b]
            pltpu.sync_copy(src_hbm.at[wid, j], src_c)
            pltpu.sync_copy(dst_hbm.at[wid, j], dst_c)
            pltpu.async_copy(tbl_hbm.at[src_c], rows_v, gsem)
            pltpu.async_copy(er_hbm.at[dst_c], er_v, esem)

        def wait(b):
            src_c, dst_c, er_v, rows_v, gsem, esem = bufs[b]
            pltpu.make_async_copy(tbl_hbm.at[src_c], rows_v, gsem).wait()
            pltpu.make_async_copy(er_hbm.at[dst_c], er_v, esem).wait()

        def compute_scatter(b):
            src_c, dst_c, er_v, rows_v, gsem, esem = bufs[b]

            def medge(i, carry):
                elv = rows_v[i, pl.ds(ecol0, 16)]
                erv = er_v[i, pl.ds(0, 16)]
                e = elv + erv
                e = jnp.where(e > 0, e, 0.2 * e)
                exm = jnp.exp(e * hmask) * hmask
                if num_heads == 4:
                    rows_v[i, pl.ds(ecol0, 16)] = exm
                    for h in range(num_heads):
                        s = exm[moff + h]
                        for q in range(dh // 16):
                            sl = pl.ds(h * dh + q * 16, 16)
                            rows_v[i, sl] = rows_v[i, sl] * s
                else:
                    s = exm[moff]
                    rows_v[i, pl.ds(0, 16)] = rows_v[i, pl.ds(0, 16)] * s
                    rows_v[i, pl.ds(16, 16)] = rows_v[i, pl.ds(16, 16)] * s
                    featmask = jnp.where(_iota16() < 8, 1.0, 0.0
                                         ).astype(jnp.float32)
                    rows_v[i, pl.ds(32, 16)] = (
                        elv * s * featmask + exm)
                return carry

            lax.fori_loop(0, CH, medge, 0)
            pltpu.async_copy(rows_v, acc_s.at[dst_c], ssem, add=True).wait()

        issue(0, 0)
        def pair_body(p, carry):
            a = 2 * p
            wait(0)
            issue(a + 1, 1)
            compute_scatter(0)
            wait(1)
            issue(a + 2, 0)
            compute_scatter(1)
            return carry

        lax.fori_loop(0, (NCH - 1) // 2, pair_body, 0)
        wait(0)
        compute_scatter(0)

        plsc.subcore_barrier()
        pltpu.sync_copy(acc_s.at[pl.ds(base, RPT)],
                        out_hbm.at[cid, pl.ds(base, RPT)])

    return edge_k


_edge_cache = {}


def _edge_kernel(tbl_w, num_heads, dh):
    key = (tbl_w, num_heads, dh)
    if key not in _edge_cache:
        _edge_cache[key] = _make_edge_kernel(tbl_w, num_heads, dh)
    return _edge_cache[key]


def _feed0_body(x_ref, wa_ref, ar_ref, tbl_ref, er_ref):
    xb = x_ref[...]
    tbl_ref[...] = jnp.dot(xb, wa_ref[...], preferred_element_type=jnp.float32)
    er_ref[...] = jnp.dot(xb, ar_ref[...], preferred_element_type=jnp.float32)


def _combine_feed_body(a_ref, b_ref, r_ref, bias_ref, wa_ref, ar_ref,
                       tbl_ref, er_ref):
    s = a_ref[...] + b_ref[...]
    den = jnp.dot(s[:, F:F + H], r_ref[...], preferred_element_type=jnp.float32)
    hb = s[:, :F] / jnp.maximum(den, 1e-9) + bias_ref[...]
    tbl_ref[...] = jnp.dot(hb, wa_ref[...], preferred_element_type=jnp.float32)
    er_ref[...] = jnp.dot(hb, ar_ref[...], preferred_element_type=jnp.float32)


def _final_body(a_ref, b_ref, bias_ref, o_ref):
    s = a_ref[...] + b_ref[...]
    den = jnp.maximum(s[:, C:C + 1], 1e-9)
    o_ref[...] = s[:, :C] / den + bias_ref[...]


def _row_spec(w):
    return pl.BlockSpec((BN, w), lambda i: (i, 0))


def _rep_spec(shape):
    return pl.BlockSpec(shape, lambda i: (0, 0))


def _feed0(x, wa, ar):
    return pl.pallas_call(
        _feed0_body,
        grid=(N // BN,),
        in_specs=[_row_spec(F), _rep_spec((F, TW01)), _rep_spec((F, 16))],
        out_specs=[_row_spec(TW01), _row_spec(16)],
        out_shape=[jax.ShapeDtypeStruct((N, TW01), jnp.float32),
                   jax.ShapeDtypeStruct((N, 16), jnp.float32)],
    )(x, wa, ar)


def _combine_feed(acc, r4, bias, wa, ar, tw):
    return pl.pallas_call(
        _combine_feed_body,
        grid=(N // BN,),
        in_specs=[_row_spec(TW01), _row_spec(TW01), _rep_spec((H, F)),
                  _rep_spec((1, F)), _rep_spec((F, tw)), _rep_spec((F, 16))],
        out_specs=[_row_spec(tw), _row_spec(16)],
        out_shape=[jax.ShapeDtypeStruct((N, tw), jnp.float32),
                   jax.ShapeDtypeStruct((N, 16), jnp.float32)],
    )(acc[0], acc[1], r4, bias, wa, ar)


def _final(acc, bias):
    return pl.pallas_call(
        _final_body,
        grid=(N // BN,),
        in_specs=[_row_spec(TW2), _row_spec(TW2), _rep_spec((1, C))],
        out_specs=_row_spec(C),
        out_shape=jax.ShapeDtypeStruct((N, C), jnp.float32),
    )(acc[0], acc[1], bias)


def _attn_mat(al, num_heads, dh):
    eye = jnp.eye(num_heads, dtype=al.dtype)
    return (al[:, :, None] * eye[:, None, :]).reshape(num_heads * dh, num_heads)


def kernel(x, edge_index, W0, al0, ar0, b0, W1, al1, ar1, b1, W2, al2, ar2, b2):
    src3 = edge_index[0].reshape(NW, NCH, CH)
    dst3 = edge_index[1].reshape(NW, NCH, CH)

    zpad01 = jnp.zeros((F, TW01 - F - H), jnp.float32)
    wa0 = jnp.concatenate([W0, W0 @ _attn_mat(al0, H, DH), zpad01], axis=1)
    wr0 = jnp.concatenate([W0 @ _attn_mat(ar0, H, DH), jnp.zeros((F, 12), jnp.float32)], axis=1)
    wa1 = jnp.concatenate([W1, W1 @ _attn_mat(al1, H, DH), zpad01], axis=1)
    wr1 = jnp.concatenate([W1 @ _attn_mat(ar1, H, DH), jnp.zeros((F, 12), jnp.float32)], axis=1)
    wa2 = jnp.concatenate([W2, W2 @ al2.T, jnp.zeros((F, TW2 - C - 1), jnp.float32)],
                          axis=1)
    wr2 = jnp.concatenate([jnp.zeros((F, 8), jnp.float32), W2 @ ar2.T,
                           jnp.zeros((F, 7), jnp.float32)], axis=1)
    r4 = jnp.kron(jnp.eye(H, dtype=jnp.float32), jnp.ones((1, DH), jnp.float32))

    edge01 = _edge_kernel(TW01, H, DH)
    edge2 = _edge_kernel(TW2, 1, C)

    tbl0, er0 = _feed0(x, wa0, wr0)
    acc0 = edge01(tbl0, er0, src3, dst3)
    tbl1, er1 = _combine_feed(acc0, r4, b0.reshape(1, F), wa1, wr1, TW01)
    acc1 = edge01(tbl1, er1, src3, dst3)
    tbl2, er2 = _combine_feed(acc1, r4, b1.reshape(1, F), wa2, wr2, TW2)
    acc2 = edge2(tbl2, er2, src3, dst3)
    return _final(acc2, b2.reshape(1, C))


# speedup vs baseline: 68.6110x; 1.4388x over previous
import functools

import jax
import jax.numpy as jnp
from jax import lax
from jax.experimental import pallas as pl
from jax.experimental.pallas import tpu as pltpu
from jax.experimental.pallas import tpu_sc as plsc

N = 10000
E = 320000
F = 128
H = 4
DH = 32
C = 40

NC = 2
NS = 16
NW = NC * NS
EPW = E // NW
CH = 80
NCH = EPW // CH
RPT = N // NS

TW01 = 144
TW2 = 48

BN = 1000


def _iota16():
    return lax.broadcasted_iota(jnp.int32, (16,), 0)


def _c16(v):
    return jnp.full((16,), v, jnp.int32)


def _make_edge_kernel(tbl_w, num_heads, dh):
    fd = num_heads * dh
    mesh = plsc.VectorSubcoreMesh(core_axis_name="c", subcore_axis_name="s",
                                  num_cores=NC, num_subcores=NS)

    @functools.partial(
        pl.kernel,
        out_type=jax.ShapeDtypeStruct((NC, N, tbl_w), jnp.float32),
        mesh=mesh,
        scratch_types=[
            pltpu.VMEM((CH,), jnp.int32),
            pltpu.VMEM((CH,), jnp.int32),
            pltpu.VMEM((CH, 16), jnp.float32),
            pltpu.VMEM((CH, tbl_w), jnp.float32),
            pltpu.VMEM((CH,), jnp.int32),
            pltpu.VMEM((CH,), jnp.int32),
            pltpu.VMEM((CH, 16), jnp.float32),
            pltpu.VMEM((CH, tbl_w), jnp.float32),
            pltpu.VMEM_SHARED((N, tbl_w), jnp.float32),
            pltpu.SemaphoreType.DMA,
            pltpu.SemaphoreType.DMA,
            pltpu.SemaphoreType.DMA,
            pltpu.SemaphoreType.DMA,
            pltpu.SemaphoreType.DMA,
        ],
        compiler_params=pltpu.CompilerParams(use_tc_tiling_on_sc=False,
                                             needs_layout_passes=False),
    )
    def edge_k(tbl_hbm, er_hbm, src_hbm, dst_hbm, out_hbm,
               src_a, dst_a, er_a, rows_a, src_b, dst_b, er_b, rows_b,
               acc_s, gsa, esa, gsb, esb, ssem):
        cid = lax.axis_index("c")
        sid = lax.axis_index("s")
        wid = cid * NS + sid
        bufs = ((src_a, dst_a, er_a, rows_a, gsa, esa),
                (src_b, dst_b, er_b, rows_b, gsb, esb))

        zero16 = jnp.zeros((16,), jnp.float32)

        def zrow(i, carry):
            for cc in range(tbl_w // 16):
                rows_a[i, pl.ds(cc * 16, 16)] = zero16
            return carry

        lax.fori_loop(0, CH, zrow, 0)

        base = sid * RPT
        nfull = RPT // CH
        rem = RPT - nfull * CH
        for k in range(nfull):
            pltpu.sync_copy(rows_a, acc_s.at[pl.ds(base + k * CH, CH)])
        pltpu.sync_copy(rows_a.at[pl.ds(0, rem)],
                        acc_s.at[pl.ds(base + nfull * CH, rem)])

        plsc.subcore_barrier()

        if num_heads == 4:
            ecol0 = fd
            moff = 0
        else:
            ecol0 = fd - 8
            moff = 8
        lanes = _iota16()
        hmask = jnp.where((lanes >= moff) & (lanes < moff + num_heads),
                          1.0, 0.0).astype(jnp.float32)

        def issue(j, b):
            src_c, dst_c, er_v, rows_v, gsem, esem = bufs[b]
            pltpu.sync_copy(src_hbm.at[wid, j], src_c)
            pltpu.sync_copy(dst_hbm.at[wid, j], dst_c)
            pltpu.async_copy(tbl_hbm.at[src_c], rows_v, gsem)
            pltpu.async_copy(er_hbm.at[dst_c], er_v, esem)

        def wait(b):
            src_c, dst_c, er_v, rows_v, gsem, esem = bufs[b]
            pltpu.make_async_copy(tbl_hbm.at[src_c], rows_v, gsem).wait()
            pltpu.make_async_copy(er_hbm.at[dst_c], er_v, esem).wait()

        def compute_scatter(b):
            src_c, dst_c, er_v, rows_v, gsem, esem = bufs[b]

            @plsc.parallel_loop(0, CH, unroll=4)
            def _(i):
                elv = rows_v[i, pl.ds(ecol0, 16)]
                erv = er_v[i, pl.ds(0, 16)]
                e = elv + erv
                e = jnp.where(e > 0, e, 0.2 * e)
                exm = jnp.exp(e * hmask) * hmask
                if num_heads == 4:
                    rows_v[i, pl.ds(ecol0, 16)] = exm
                    for h in range(num_heads):
                        s = exm[moff + h]
                        for q in range(dh // 16):
                            sl = pl.ds(h * dh + q * 16, 16)
                            rows_v[i, sl] = rows_v[i, sl] * s
                else:
                    s = exm[moff]
                    rows_v[i, pl.ds(0, 16)] = rows_v[i, pl.ds(0, 16)] * s
                    rows_v[i, pl.ds(16, 16)] = rows_v[i, pl.ds(16, 16)] * s
                    featmask = jnp.where(_iota16() < 8, 1.0, 0.0
                                         ).astype(jnp.float32)
                    rows_v[i, pl.ds(32, 16)] = (
                        elv * s * featmask + exm)

            pltpu.async_copy(rows_v, acc_s.at[dst_c], ssem, add=True).wait()

        issue(0, 0)
        def pair_body(p, carry):
            a = 2 * p
            wait(0)
            issue(a + 1, 1)
            compute_scatter(0)
            wait(1)
            issue(a + 2, 0)
            compute_scatter(1)
            return carry

        lax.fori_loop(0, (NCH - 1) // 2, pair_body, 0)
        wait(0)
        compute_scatter(0)

        plsc.subcore_barrier()
        pltpu.sync_copy(acc_s.at[pl.ds(base, RPT)],
                        out_hbm.at[cid, pl.ds(base, RPT)])

    return edge_k


_edge_cache = {}


def _edge_kernel(tbl_w, num_heads, dh):
    key = (tbl_w, num_heads, dh)
    if key not in _edge_cache:
        _edge_cache[key] = _make_edge_kernel(tbl_w, num_heads, dh)
    return _edge_cache[key]


def _feed0_body(x_ref, wa_ref, ar_ref, tbl_ref, er_ref):
    xb = x_ref[...]
    tbl_ref[...] = jnp.dot(xb, wa_ref[...], preferred_element_type=jnp.float32)
    er_ref[...] = jnp.dot(xb, ar_ref[...], preferred_element_type=jnp.float32)


def _combine_feed_body(a_ref, b_ref, r_ref, bias_ref, wa_ref, ar_ref,
                       tbl_ref, er_ref):
    s = a_ref[...] + b_ref[...]
    den = jnp.dot(s[:, F:F + H], r_ref[...], preferred_element_type=jnp.float32)
    hb = s[:, :F] / jnp.maximum(den, 1e-9) + bias_ref[...]
    tbl_ref[...] = jnp.dot(hb, wa_ref[...], preferred_element_type=jnp.float32)
    er_ref[...] = jnp.dot(hb, ar_ref[...], preferred_element_type=jnp.float32)


def _final_body(a_ref, b_ref, bias_ref, o_ref):
    s = a_ref[...] + b_ref[...]
    den = jnp.maximum(s[:, C:C + 1], 1e-9)
    o_ref[...] = s[:, :C] / den + bias_ref[...]


def _row_spec(w):
    return pl.BlockSpec((BN, w), lambda i: (i, 0))


def _rep_spec(shape):
    return pl.BlockSpec(shape, lambda i: (0, 0))


def _feed0(x, wa, ar):
    return pl.pallas_call(
        _feed0_body,
        grid=(N // BN,),
        in_specs=[_row_spec(F), _rep_spec((F, TW01)), _rep_spec((F, 16))],
        out_specs=[_row_spec(TW01), _row_spec(16)],
        out_shape=[jax.ShapeDtypeStruct((N, TW01), jnp.float32),
                   jax.ShapeDtypeStruct((N, 16), jnp.float32)],
    )(x, wa, ar)


def _combine_feed(acc, r4, bias, wa, ar, tw):
    return pl.pallas_call(
        _combine_feed_body,
        grid=(N // BN,),
        in_specs=[_row_spec(TW01), _row_spec(TW01), _rep_spec((H, F)),
                  _rep_spec((1, F)), _rep_spec((F, tw)), _rep_spec((F, 16))],
        out_specs=[_row_spec(tw), _row_spec(16)],
        out_shape=[jax.ShapeDtypeStruct((N, tw), jnp.float32),
                   jax.ShapeDtypeStruct((N, 16), jnp.float32)],
    )(acc[0], acc[1], r4, bias, wa, ar)


def _final(acc, bias):
    return pl.pallas_call(
        _final_body,
        grid=(N // BN,),
        in_specs=[_row_spec(TW2), _row_spec(TW2), _rep_spec((1, C))],
        out_specs=_row_spec(C),
        out_shape=jax.ShapeDtypeStruct((N, C), jnp.float32),
    )(acc[0], acc[1], bias)


def _attn_mat(al, num_heads, dh):
    eye = jnp.eye(num_heads, dtype=al.dtype)
    return (al[:, :, None] * eye[:, None, :]).reshape(num_heads * dh, num_heads)


def kernel(x, edge_index, W0, al0, ar0, b0, W1, al1, ar1, b1, W2, al2, ar2, b2):
    src3 = edge_index[0].reshape(NW, NCH, CH)
    dst3 = edge_index[1].reshape(NW, NCH, CH)

    zpad01 = jnp.zeros((F, TW01 - F - H), jnp.float32)
    wa0 = jnp.concatenate([W0, W0 @ _attn_mat(al0, H, DH), zpad01], axis=1)
    wr0 = jnp.concatenate([W0 @ _attn_mat(ar0, H, DH), jnp.zeros((F, 12), jnp.float32)], axis=1)
    wa1 = jnp.concatenate([W1, W1 @ _attn_mat(al1, H, DH), zpad01], axis=1)
    wr1 = jnp.concatenate([W1 @ _attn_mat(ar1, H, DH), jnp.zeros((F, 12), jnp.float32)], axis=1)
    wa2 = jnp.concatenate([W2, W2 @ al2.T, jnp.zeros((F, TW2 - C - 1), jnp.float32)],
                          axis=1)
    wr2 = jnp.concatenate([jnp.zeros((F, 8), jnp.float32), W2 @ ar2.T,
                           jnp.zeros((F, 7), jnp.float32)], axis=1)
    r4 = jnp.kron(jnp.eye(H, dtype=jnp.float32), jnp.ones((1, DH), jnp.float32))

    edge01 = _edge_kernel(TW01, H, DH)
    edge2 = _edge_kernel(TW2, 1, C)

    tbl0, er0 = _feed0(x, wa0, wr0)
    acc0 = edge01(tbl0, er0, src3, dst3)
    tbl1, er1 = _combine_feed(acc0, r4, b0.reshape(1, F), wa1, wr1, TW01)
    acc1 = edge01(tbl1, er1, src3, dst3)
    tbl2, er2 = _combine_feed(acc1, r4, b1.reshape(1, F), wa2, wr2, TW2)
    acc2 = edge2(tbl2, er2, src3, dst3)
    return _final(acc2, b2.reshape(1, C))
